# baseline (speedup 1.0000x reference)
import functools
import jax
import jax.numpy as jnp
from jax import lax
from jax.experimental import pallas as pl
from jax.experimental.pallas import tpu as pltpu
from jax.experimental.pallas import tpu_sc as plsc

_CHUNK = 32


def _sc_kernel(x, pos_table):
    B, S, D = x.shape
    NC, NS = 2, 16
    NW = NC * NS
    rows_per_w = S // NW
    n_chunks = rows_per_w // _CHUNK
    cd = _CHUNK * D
    n_vec = cd // 16
    x1 = x.reshape(B * S * D)
    t1 = pos_table.reshape(pos_table.shape[0] * D)
    mesh = plsc.VectorSubcoreMesh(
        core_axis_name="c", subcore_axis_name="s", num_cores=NC
    )

    @functools.partial(
        pl.kernel,
        mesh=mesh,
        out_type=jax.ShapeDtypeStruct((B * S * D,), jnp.float32),
        scratch_types=[
            pltpu.VMEM((cd,), jnp.float32),
            pltpu.VMEM((cd,), jnp.float32),
        ],
    )
    def k(x_hbm, tbl_hbm, out_hbm, tbl_v, acc_v):
        wid = lax.axis_index("s") * NC + lax.axis_index("c")
        row0 = wid * rows_per_w
        for c in range(n_chunks):
            base = (row0 + c * _CHUNK) * D
            pltpu.sync_copy(tbl_hbm.at[pl.ds(base, cd)], tbl_v)
            for b in range(B):
                off = b * S * D + base
                pltpu.sync_copy(x_hbm.at[pl.ds(off, cd)], acc_v)

                def body(i, _):
                    sl = pl.ds(i * 16, 16)
                    acc_v[sl] = acc_v[sl] + tbl_v[sl]
                    return 0

                lax.fori_loop(0, n_vec, body, 0)
                pltpu.sync_copy(acc_v, out_hbm.at[pl.ds(off, cd)])

    out1 = k(x1, t1)
    return out1.reshape(B, S, D)


def kernel(x, pos_table):
    return _sc_kernel(x, pos_table)


# speedup vs baseline: 7.8062x; 7.8062x over previous
import jax
import jax.numpy as jnp
from jax.experimental import pallas as pl


def _add_kernel(x_ref, pos_ref, o_ref):
    o_ref[...] = x_ref[...] + pos_ref[...]


def kernel(x, pos_table):
    B, S, D = x.shape
    CS = 2048
    grid = (S // CS, B)
    return pl.pallas_call(
        _add_kernel,
        grid=grid,
        in_specs=[
            pl.BlockSpec((1, CS, D), lambda s, b: (b, s, 0)),
            pl.BlockSpec((CS, D), lambda s, b: (s, 0)),
        ],
        out_specs=pl.BlockSpec((1, CS, D), lambda s, b: (b, s, 0)),
        out_shape=jax.ShapeDtypeStruct((B, S, D), x.dtype),
    )(x, pos_table)
